# Initial kernel scaffold; baseline (speedup 1.0000x reference)
import jax
import jax.numpy as jnp
from jax.experimental import pallas as pl


def kernel(x, edge_index, W0_1, W1_1, b1, W0_2, W1_2, b2):
    raise NotImplementedError("write your pallas kernel here")


# speedup vs baseline: 18.3679x; 18.3679x over previous
import functools

import jax
import jax.numpy as jnp
from jax import lax
from jax.experimental import pallas as pl
from jax.experimental.pallas import tpu as pltpu
from jax.experimental.pallas import tpu_sc as plsc

NC = 2
NS = 16
NW = NC * NS
L = 16
CH = 128


def _make_sc_deg(NP, CPT):
    mesh = plsc.VectorSubcoreMesh(core_axis_name="c", subcore_axis_name="s")

    @functools.partial(
        pl.kernel,
        out_type=jax.ShapeDtypeStruct((NW * NP,), jnp.float32),
        mesh=mesh,
        scratch_types=[
            pltpu.VMEM((CPT * CH,), jnp.int32),
            pltpu.VMEM((NP,), jnp.float32),
        ],
        compiler_params=pltpu.CompilerParams(needs_layout_passes=False),
    )
    def deg_kernel(src_hbm, degp_hbm, idx_v, deg_v):
        c = lax.axis_index("c")
        s = lax.axis_index("s")
        wid = s * NC + c

        def zero_body(i, _):
            deg_v[pl.ds(i * L, L)] = jnp.zeros((L,), jnp.float32)
            return 0

        lax.fori_loop(0, NP // L, zero_body, 0)

        pltpu.sync_copy(src_hbm.at[pl.ds(wid * CPT * CH, CPT * CH)], idx_v)

        ones = jnp.ones((L,), jnp.float32)

        def body(j, _):
            for i in range(CH // L):
                iv = idx_v[pl.ds(j * CH + i * L, L)]
                plsc.addupdate_scatter(deg_v, [iv], ones)
            return 0

        lax.fori_loop(0, CPT, body, 0)
        pltpu.sync_copy(deg_v, degp_hbm.at[pl.ds(wid * NP, NP)])

    return deg_kernel


def _make_sc_segsum(NP, CPT):
    mesh = plsc.VectorSubcoreMesh(core_axis_name="c", subcore_axis_name="s")
    RPT = NP // NS

    @functools.partial(
        pl.kernel,
        out_type=jax.ShapeDtypeStruct((NC, NP, L), jnp.float32),
        mesh=mesh,
        scratch_types=[
            pltpu.VMEM((CH,), jnp.int32),
            pltpu.VMEM((CH,), jnp.int32),
            pltpu.VMEM((CH, L), jnp.float32),
            pltpu.VMEM_SHARED((NP, L), jnp.float32),
            pltpu.SemaphoreType.DMA,
        ],
        compiler_params=pltpu.CompilerParams(use_tc_tiling_on_sc=False),
    )
    def seg_kernel(z_hbm, src_hbm, dst_hbm, zero_hbm, sp_hbm, idx_s, idx_d,
                   rows_v, acc, sem):
        c = lax.axis_index("c")
        s = lax.axis_index("s")
        wid = s * NC + c

        pltpu.sync_copy(zero_hbm.at[pl.ds(s * RPT, RPT)],
                        acc.at[pl.ds(s * RPT, RPT)])
        plsc.subcore_barrier()

        def body(j, _):
            base = (wid * CPT + j) * CH
            pltpu.sync_copy(src_hbm.at[pl.ds(base, CH)], idx_s)
            pltpu.sync_copy(dst_hbm.at[pl.ds(base, CH)], idx_d)
            pltpu.async_copy(z_hbm.at[idx_s], rows_v, sem).wait()
            pltpu.sync_copy(rows_v, acc.at[idx_d], add=True)
            return 0

        lax.fori_loop(0, CPT, body, 0)
        plsc.subcore_barrier()
        pltpu.sync_copy(acc.at[pl.ds(s * RPT, RPT)],
                        sp_hbm.at[c, pl.ds(s * RPT, RPT)])

    return seg_kernel


def _tc_b(x_p, W0_1, W1_1, degp, NP):
    def body(x_ref, w0_ref, w1_ref, degp_ref, z1_ref, xw0_ref, dinv_ref):
        deg = jnp.sum(degp_ref[...].reshape(NW, NP), axis=0)
        dinv = jnp.where(deg > 0.0, lax.rsqrt(deg), 0.0)
        xv = x_ref[...]
        y1 = jnp.dot(xv, w1_ref[...], preferred_element_type=jnp.float32)
        z1_ref[...] = y1 * dinv[:, None]
        xw0_ref[...] = jnp.dot(xv, w0_ref[...],
                               preferred_element_type=jnp.float32)
        dinv_ref[...] = dinv

    return pl.pallas_call(
        body,
        out_shape=(
            jax.ShapeDtypeStruct((NP, L), jnp.float32),
            jax.ShapeDtypeStruct((NP, L), jnp.float32),
            jax.ShapeDtypeStruct((NP,), jnp.float32),
        ),
    )(x_p, W0_1, W1_1, degp)


def _tc_d(xw0, s1p, dinv, b1, NP):
    def body(xw0_ref, s1p_ref, dinv_ref, b1_ref, h_ref, z2_ref):
        s1 = s1p_ref[0] + s1p_ref[1]
        dv = dinv_ref[...][:, None]
        h = jnp.maximum(xw0_ref[...] - dv * s1 + b1_ref[...], 0.0)
        h_ref[...] = h
        z2_ref[...] = dv * h

    return pl.pallas_call(
        body,
        out_shape=(
            jax.ShapeDtypeStruct((NP, L), jnp.float32),
            jax.ShapeDtypeStruct((NP, L), jnp.float32),
        ),
    )(xw0, s1p, dinv, b1.reshape(1, L))


def _tc_f(h, s2p, dinv, W0_2, W1_2, b2, NP, D_out):
    def body(h_ref, s2p_ref, dinv_ref, w0_ref, w1_ref, b2_ref, out_ref):
        dv = dinv_ref[...][:, None]
        t = -dv * (s2p_ref[0] + s2p_ref[1])
        out_ref[...] = (
            jnp.dot(h_ref[...], w0_ref[...], preferred_element_type=jnp.float32)
            + jnp.dot(t, w1_ref[...], preferred_element_type=jnp.float32)
            + b2_ref[...]
        )

    return pl.pallas_call(
        body,
        out_shape=jax.ShapeDtypeStruct((NP, D_out), jnp.float32),
    )(h, s2p, dinv, W0_2, W1_2, b2.reshape(1, D_out))


def kernel(x, edge_index, W0_1, W1_1, b1, W0_2, W1_2, b2):
    N, _ = x.shape
    E = edge_index.shape[1]
    D_out = W0_2.shape[1]

    NP = ((N + 1 + 1279) // 1280) * 1280
    CPT = (-(-E // (NW * CH)) + 7) // 8 * 8
    EP = NW * CPT * CH

    src = edge_index[0].astype(jnp.int32)
    dst = edge_index[1].astype(jnp.int32)
    pad = jnp.full((EP - E,), N, jnp.int32)
    src_flat = jnp.concatenate([src, pad])
    dst_flat = jnp.concatenate([dst, pad])
    x_p = jnp.concatenate(
        [x, jnp.zeros((NP - N, x.shape[1]), jnp.float32)], axis=0)
    zeros_nl = jnp.zeros((NP, L), jnp.float32)

    degp = _make_sc_deg(NP, CPT)(src_flat)
    z1, xw0, dinv = _tc_b(x_p, W0_1, W1_1, degp, NP)
    seg = _make_sc_segsum(NP, CPT)
    s1p = seg(z1, src_flat, dst_flat, zeros_nl)
    h, z2 = _tc_d(xw0, s1p, dinv, b1, NP)
    s2p = seg(z2, src_flat, dst_flat, zeros_nl)
    out = _tc_f(h, s2p, dinv, W0_2, W1_2, b2, NP, D_out)
    return out[:N]


# speedup vs baseline: 33.1157x; 1.8029x over previous
import functools

import jax
import jax.numpy as jnp
from jax import lax
from jax.experimental import pallas as pl
from jax.experimental.pallas import tpu as pltpu
from jax.experimental.pallas import tpu_sc as plsc

NC = 2
NS = 16
NW = NC * NS
L = 16
CH = 128


def _make_sc_deg(NP, CPT):
    mesh = plsc.VectorSubcoreMesh(core_axis_name="c", subcore_axis_name="s")

    @functools.partial(
        pl.kernel,
        out_type=jax.ShapeDtypeStruct((NW * NP,), jnp.float32),
        mesh=mesh,
        scratch_types=[
            pltpu.VMEM((CPT * CH,), jnp.int32),
            pltpu.VMEM((NP,), jnp.float32),
        ],
        compiler_params=pltpu.CompilerParams(needs_layout_passes=False),
    )
    def deg_kernel(src_hbm, degp_hbm, idx_v, deg_v):
        c = lax.axis_index("c")
        s = lax.axis_index("s")
        wid = s * NC + c

        def zero_body(i, _):
            deg_v[pl.ds(i * L, L)] = jnp.zeros((L,), jnp.float32)
            return 0

        lax.fori_loop(0, NP // L, zero_body, 0)

        pltpu.sync_copy(src_hbm.at[pl.ds(wid * CPT * CH, CPT * CH)], idx_v)

        ones = jnp.ones((L,), jnp.float32)

        def body(j, _):
            for i in range(CH // L):
                iv = idx_v[pl.ds(j * CH + i * L, L)]
                plsc.addupdate_scatter(deg_v, [iv], ones)
            return 0

        lax.fori_loop(0, CPT, body, 0)
        pltpu.sync_copy(deg_v, degp_hbm.at[pl.ds(wid * NP, NP)])

    return deg_kernel


def _make_sc_segsum(NP, CPT):
    mesh = plsc.VectorSubcoreMesh(core_axis_name="c", subcore_axis_name="s")
    RPT = NP // NS
    G = 8
    NG = CPT // G
    NSLOT = 2 * G

    scratch = [pltpu.VMEM((CH,), jnp.int32) for _ in range(NSLOT)]
    scratch += [pltpu.VMEM((CH,), jnp.int32) for _ in range(NSLOT)]
    scratch += [pltpu.VMEM((CH, L), jnp.float32) for _ in range(NSLOT)]
    scratch += [
        pltpu.VMEM_SHARED((NP, L), jnp.float32),
        pltpu.SemaphoreType.DMA,
        pltpu.SemaphoreType.DMA,
        pltpu.SemaphoreType.DMA,
    ]

    @functools.partial(
        pl.kernel,
        out_type=jax.ShapeDtypeStruct((NC, NP, L), jnp.float32),
        mesh=mesh,
        scratch_types=scratch,
        compiler_params=pltpu.CompilerParams(use_tc_tiling_on_sc=False),
    )
    def seg_kernel(z_hbm, src_hbm, dst_hbm, zero_hbm, sp_hbm, *scr):
        sbufs = scr[0:NSLOT]
        dbufs = scr[NSLOT:2 * NSLOT]
        rbufs = scr[2 * NSLOT:3 * NSLOT]
        acc, semi, semg, sems = scr[3 * NSLOT:]
        c = lax.axis_index("c")
        s = lax.axis_index("s")
        wid = s * NC + c

        pltpu.sync_copy(zero_hbm.at[pl.ds(s * RPT, RPT)],
                        acc.at[pl.ds(s * RPT, RPT)])
        plsc.subcore_barrier()

        def idxcp(j, slot):
            base = (wid * CPT + j) * CH
            pltpu.async_copy(src_hbm.at[pl.ds(base, CH)], sbufs[slot], semi)
            return pltpu.async_copy(dst_hbm.at[pl.ds(base, CH)],
                                    dbufs[slot], semi)

        def gather(slot):
            return pltpu.async_copy(z_hbm.at[sbufs[slot]], rbufs[slot], semg)

        def scatter(slot):
            pltpu.sync_copy(rbufs[slot], acc.at[dbufs[slot]], add=True)

        ids, gds = {}, {}
        for b in range(G):
            ids[b] = idxcp(b, b)
        for b in range(G):
            ids[b].wait()
            ids[b].wait()
            gds[b] = gather(b)
        for g in range(NG):
            base = g * G
            for b in range(G):
                gds[base + b].wait()
            if g + 1 < NG:
                for b in range(G):
                    j = (g + 1) * G + b
                    slot = ((g + 1) % 2) * G + b
                    ids[j] = idxcp(j, slot)
                for b in range(G):
                    j = (g + 1) * G + b
                    slot = ((g + 1) % 2) * G + b
                    ids[j].wait()
                    ids[j].wait()
                    gds[j] = gather(slot)
            for b in range(G):
                scatter((g % 2) * G + b)

        plsc.subcore_barrier()
        pltpu.sync_copy(acc.at[pl.ds(s * RPT, RPT)],
                        sp_hbm.at[c, pl.ds(s * RPT, RPT)])

    return seg_kernel


def _tc_b(x_p, W0_1, W1_1, degp, NP):
    def body(x_ref, w0_ref, w1_ref, degp_ref, z1_ref, xw0_ref, dinv_ref):
        deg = jnp.sum(degp_ref[...].reshape(NW, NP), axis=0)
        dinv = jnp.where(deg > 0.0, lax.rsqrt(deg), 0.0)
        xv = x_ref[...]
        y1 = jnp.dot(xv, w1_ref[...], preferred_element_type=jnp.float32)
        z1_ref[...] = y1 * dinv[:, None]
        xw0_ref[...] = jnp.dot(xv, w0_ref[...],
                               preferred_element_type=jnp.float32)
        dinv_ref[...] = dinv

    return pl.pallas_call(
        body,
        out_shape=(
            jax.ShapeDtypeStruct((NP, L), jnp.float32),
            jax.ShapeDtypeStruct((NP, L), jnp.float32),
            jax.ShapeDtypeStruct((NP,), jnp.float32),
        ),
    )(x_p, W0_1, W1_1, degp)


def _tc_d(xw0, s1p, dinv, b1, NP):
    def body(xw0_ref, s1p_ref, dinv_ref, b1_ref, h_ref, z2_ref):
        s1 = s1p_ref[0] + s1p_ref[1]
        dv = dinv_ref[...][:, None]
        h = jnp.maximum(xw0_ref[...] - dv * s1 + b1_ref[...], 0.0)
        h_ref[...] = h
        z2_ref[...] = dv * h

    return pl.pallas_call(
        body,
        out_shape=(
            jax.ShapeDtypeStruct((NP, L), jnp.float32),
            jax.ShapeDtypeStruct((NP, L), jnp.float32),
        ),
    )(xw0, s1p, dinv, b1.reshape(1, L))


def _tc_f(h, s2p, dinv, W0_2, W1_2, b2, NP, D_out):
    def body(h_ref, s2p_ref, dinv_ref, w0_ref, w1_ref, b2_ref, out_ref):
        dv = dinv_ref[...][:, None]
        t = -dv * (s2p_ref[0] + s2p_ref[1])
        out_ref[...] = (
            jnp.dot(h_ref[...], w0_ref[...], preferred_element_type=jnp.float32)
            + jnp.dot(t, w1_ref[...], preferred_element_type=jnp.float32)
            + b2_ref[...]
        )

    return pl.pallas_call(
        body,
        out_shape=jax.ShapeDtypeStruct((NP, D_out), jnp.float32),
    )(h, s2p, dinv, W0_2, W1_2, b2.reshape(1, D_out))


def kernel(x, edge_index, W0_1, W1_1, b1, W0_2, W1_2, b2):
    N, _ = x.shape
    E = edge_index.shape[1]
    D_out = W0_2.shape[1]

    NP = ((N + 1 + 1279) // 1280) * 1280
    CPT = (-(-E // (NW * CH)) + 7) // 8 * 8
    EP = NW * CPT * CH

    src = edge_index[0].astype(jnp.int32)
    dst = edge_index[1].astype(jnp.int32)
    pad = jnp.full((EP - E,), N, jnp.int32)
    src_flat = jnp.concatenate([src, pad])
    dst_flat = jnp.concatenate([dst, pad])
    src2d = src_flat.reshape(EP // CH, CH)
    x_p = jnp.concatenate(
        [x, jnp.zeros((NP - N, x.shape[1]), jnp.float32)], axis=0)
    zeros_nl = jnp.zeros((NP, L), jnp.float32)
    del src2d

    degp = _make_sc_deg(NP, CPT)(src_flat)
    z1, xw0, dinv = _tc_b(x_p, W0_1, W1_1, degp, NP)
    seg = _make_sc_segsum(NP, CPT)
    s1p = seg(z1, src_flat, dst_flat, zeros_nl)
    h, z2 = _tc_d(xw0, s1p, dinv, b1, NP)
    s2p = seg(z2, src_flat, dst_flat, zeros_nl)
    out = _tc_f(h, s2p, dinv, W0_2, W1_2, b2, NP, D_out)
    return out[:N]
